# Initial kernel scaffold; baseline (speedup 1.0000x reference)
import jax
import jax.numpy as jnp
from jax.experimental import pallas as pl


def kernel(feat, edge_index, mask_nodes, W1, b1, g1, be1, a1, W2, b2, g2, be2, a2, tW1, tb1, tg1, tbe1, ta1, tW2, tb2, tg2, tbe2, ta2, dW, db, dg, dbe, da, mask_token, pW1, pb1, pW2, pb2, qW1, qb1, qW2, qb2):
    raise NotImplementedError("write your pallas kernel here")


# speedup vs baseline: 7.1446x; 7.1446x over previous
import functools

import jax
import jax.numpy as jnp
from jax import lax
from jax.experimental import pallas as pl
from jax.experimental.pallas import tpu as pltpu
from jax.experimental.pallas import tpu_sc as plsc

N = 10000
E = 320000
D = 128
H = 256
T = 0.2
ALPHA = 0.5

NC, NS = 2, 16
NW = NC * NS
EPT = E // NW
KW = 125
NWIN = EPT // KW
MN = 5000
MP = 5120
MPT = MP // NW
MKW = 80
NH = 10240
HPT = NH // NS
NA = 10240
APT = NA // NS
FL = 128

_MESH = dict(core_axis_name="c", subcore_axis_name="s")


def _wid():
    return lax.axis_index("s") * NC + lax.axis_index("c")


def _hist_body(src3, dst3, msk3, mupd3, ones_h, z_h,
               degs_o, degd_o, m01_o,
               sidx_v, didx_v, midx_v, mupd_v, ones_v, z_v, bounce_v,
               hs_sh, hd_sh, hm_sh):
    core = lax.axis_index("c")
    sid = lax.axis_index("s")
    wid = _wid()
    pltpu.sync_copy(z_h, z_v)
    pltpu.sync_copy(z_v, hs_sh.at[pl.ds(sid * HPT, HPT)])
    pltpu.sync_copy(z_v, hd_sh.at[pl.ds(sid * HPT, HPT)])
    pltpu.sync_copy(z_v, hm_sh.at[pl.ds(sid * HPT, HPT)])
    pltpu.sync_copy(ones_h, ones_v)
    pltpu.sync_copy(src3.at[wid], sidx_v)
    pltpu.sync_copy(dst3.at[wid], didx_v)
    pltpu.sync_copy(msk3.at[wid], midx_v)
    pltpu.sync_copy(mupd3.at[wid], mupd_v)
    plsc.subcore_barrier()

    def win(j, carry):
        pltpu.sync_copy(ones_v, hs_sh.at[sidx_v.at[j]], add=True)
        pltpu.sync_copy(ones_v, hd_sh.at[didx_v.at[j]], add=True)
        return carry

    lax.fori_loop(0, NWIN, win, 0)
    pltpu.sync_copy(mupd_v.at[0], hm_sh.at[midx_v.at[0]], add=True)
    pltpu.sync_copy(mupd_v.at[1], hm_sh.at[midx_v.at[1]], add=True)
    plsc.subcore_barrier()

    @pl.when(sid == 0)
    def _f0():
        pltpu.sync_copy(hs_sh, bounce_v)
        pltpu.sync_copy(bounce_v, degs_o.at[core])

    @pl.when(sid == 1)
    def _f1():
        pltpu.sync_copy(hd_sh, bounce_v)
        pltpu.sync_copy(bounce_v, degd_o.at[core])

    @pl.when(sid == 2)
    def _f2():
        pltpu.sync_copy(hm_sh, bounce_v)
        pltpu.sync_copy(bounce_v, m01_o.at[core])


@functools.cache
def _hist_kernel():
    return pl.kernel(
        _hist_body,
        out_type=(
            jax.ShapeDtypeStruct((NC, NH), jnp.float32),
            jax.ShapeDtypeStruct((NC, NH), jnp.float32),
            jax.ShapeDtypeStruct((NC, NH), jnp.float32),
        ),
        mesh=plsc.VectorSubcoreMesh(**_MESH),
        scratch_types=(
            pltpu.VMEM((NWIN, KW), jnp.int32),
            pltpu.VMEM((NWIN, KW), jnp.int32),
            pltpu.VMEM((MPT // MKW, MKW), jnp.int32),
            pltpu.VMEM((MPT // MKW, MKW), jnp.float32),
            pltpu.VMEM((KW,), jnp.float32),
            pltpu.VMEM((HPT,), jnp.float32),
            pltpu.VMEM((NH,), jnp.float32),
            pltpu.VMEM_SHARED((NH,), jnp.float32),
            pltpu.VMEM_SHARED((NH,), jnp.float32),
            pltpu.VMEM_SHARED((NH,), jnp.float32),
        ),
    )


def _make_prop(nchunks, ngather):
    def body(*refs):
        ys = refs[:nchunks]
        src3, dst3, z_h = refs[nchunks:nchunks + 3]
        k = nchunks + 3
        gidx_h = None
        gts = ()
        if ngather:
            gidx_h = refs[k]
            gts = refs[k + 1:k + 1 + ngather]
            k += 1 + ngather
        outs = refs[k:k + nchunks]
        k += nchunks
        gouts = refs[k:k + ngather]
        k += ngather
        sidx_v, didx_v, wbuf_v, sem = refs[k:k + 4]
        if ngather:
            gidx_v = refs[k + 4]
        acc_sh = refs[-1]

        core = lax.axis_index("c")
        sid = lax.axis_index("s")
        wid = _wid()
        pltpu.sync_copy(src3.at[wid], sidx_v)
        pltpu.sync_copy(dst3.at[wid], didx_v)
        for c in range(nchunks):
            pltpu.sync_copy(z_h, wbuf_v)
            for r in range(APT // FL):
                pltpu.sync_copy(
                    wbuf_v, acc_sh.at[pl.ds(sid * APT + r * FL, FL)])
            plsc.subcore_barrier()
            win_dst = wbuf_v.at[pl.ds(0, KW)]

            def win(j, carry):
                pltpu.async_copy(ys[c].at[sidx_v.at[j]], win_dst, sem).wait()
                pltpu.sync_copy(win_dst, acc_sh.at[didx_v.at[j]], add=True)
                return carry

            lax.fori_loop(0, NWIN, win, 0)
            plsc.subcore_barrier()
            for r in range(APT // FL):
                rows = pl.ds(sid * APT + r * FL, FL)
                pltpu.sync_copy(acc_sh.at[rows], wbuf_v)
                pltpu.sync_copy(wbuf_v, outs[c].at[core].at[rows])
            plsc.subcore_barrier()
        if ngather:
            pltpu.sync_copy(gidx_h.at[pl.ds(wid * MPT, MPT)], gidx_v)
            gwin = wbuf_v.at[pl.ds(0, MKW)]
            for t in range(ngather):
                for j in range(MPT // MKW):
                    pltpu.async_copy(
                        gts[t].at[gidx_v.at[pl.ds(j * MKW, MKW)]], gwin, sem
                    ).wait()
                    pltpu.sync_copy(
                        gwin, gouts[t].at[pl.ds(wid * MPT + j * MKW, MKW)])

    out_type = tuple(
        jax.ShapeDtypeStruct((NC, NA, D), jnp.float32) for _ in range(nchunks)
    ) + tuple(
        jax.ShapeDtypeStruct((MP, D), jnp.float32) for _ in range(ngather)
    )
    scratch = [
        pltpu.VMEM((NWIN, KW), jnp.int32),
        pltpu.VMEM((NWIN, KW), jnp.int32),
        pltpu.VMEM((FL, D), jnp.float32),
        pltpu.SemaphoreType.DMA,
    ]
    if ngather:
        scratch.append(pltpu.VMEM((MPT,), jnp.int32))
    scratch.append(pltpu.VMEM_SHARED((NA, D), jnp.float32))
    return pl.kernel(
        body,
        out_type=out_type,
        mesh=plsc.VectorSubcoreMesh(**_MESH),
        scratch_types=tuple(scratch),
    )


_make_prop = functools.cache(_make_prop)


def _hist_call(*args):
    return _hist_kernel()(*args)


def _prop2(*args):
    return _make_prop(2, 0)(*args)


def _prop1g2(*args):
    return _make_prop(1, 2)(*args)


def _bn(x, g, b):
    m = x.mean(0)
    v = x.var(0)
    return (x - m) / jnp.sqrt(v + 1e-5) * g + b


def _prelu(x, a):
    return jnp.where(x >= 0, x, a * x)


def _l2n(x):
    return x / jnp.maximum(jnp.linalg.norm(x, axis=-1, keepdims=True), 1e-12)


def kernel(feat, edge_index, mask_nodes, W1, b1, g1, be1, a1, W2, b2, g2, be2,
           a2, tW1, tb1, tg1, tbe1, ta1, tW2, tb2, tg2, tbe2, ta2,
           dW, db, dg, dbe, da, mask_token,
           pW1, pb1, pW2, pb2, qW1, qb1, qW2, qb2):
    src3 = edge_index[0].reshape(NW, NWIN, KW)
    dst3 = edge_index[1].reshape(NW, NWIN, KW)
    mpad = jnp.concatenate(
        [mask_nodes, jnp.zeros((MP - MN,), jnp.int32)])
    msk3 = mpad.reshape(NW, MPT // MKW, MKW)
    mupd3 = jnp.concatenate(
        [jnp.ones((MN,), jnp.float32), jnp.zeros((MP - MN,), jnp.float32)]
    ).reshape(NW, MPT // MKW, MKW)
    ones_h = jnp.ones((KW,), jnp.float32)
    zh_h = jnp.zeros((HPT,), jnp.float32)
    zr_h = jnp.zeros((FL, D), jnp.float32)

    degs2, degd2, m012 = _hist_call(src3, dst3, msk3, mupd3, ones_h, zh_h)
    ns = jnp.clip(degs2[0, :N] + degs2[1, :N], 1.0, None) ** -0.5
    nd = jnp.clip(degd2[0, :N] + degd2[1, :N], 1.0, None) ** -0.5
    m01 = (m012[0, :N] + m012[1, :N])[:, None]

    x = feat * (1.0 - m01) + m01 * mask_token
    y0 = x * ns[:, None]
    y1 = feat * ns[:, None]
    p0, p1 = _prop2(y0, y1, src3, dst3, zr_h)

    def post(p, W, b, g, be, a):
        z = nd[:, None] * (p[0, :N] + p[1, :N])
        return _prelu(_bn(z @ W + b, g, be), a)

    e1 = post(p0, W1, b1, g1, be1, a1)
    te1 = post(p1, tW1, tb1, tg1, tbe1, ta1)

    y2a = (e1 * ns[:, None]) @ W2
    y2b = (te1 * ns[:, None]) @ tW2
    q0, q1 = _prop2(y2a, y2b, src3, dst3, zr_h)

    def post2(p, b, g, be, a):
        z = nd[:, None] * (p[0, :N] + p[1, :N])
        return _prelu(_bn(z + b, g, be), a)

    o = post2(q0, b2, g2, be2, a2)
    h2 = post2(q1, tb2, tg2, tbe2, ta2)

    y3 = ((o * (1.0 - m01)) * ns[:, None]) @ dW
    r0, om, hm = _prop1g2(y3, src3, dst3, zr_h, mpad, o, h2)
    re_x = _prelu(_bn(nd[:, None] * (r0[0, :N] + r0[1, :N]) + db, dg, dbe), da)

    rn = _l2n(re_x)
    fn = _l2n(feat)
    cos = (rn * fn).sum(-1)
    loss1 = (m01[:, 0] * (1.0 - cos)).sum() / MN

    ch = jax.nn.relu(hm @ pW1 + pb1) @ pW2 + pb2
    cm = jax.nn.relu(om @ qW1 + qb1) @ qW2 + qb2
    nh = _l2n(ch)
    nm = _l2n(cm)
    sim = jnp.exp((nh @ nm.T) / T)
    colmask = (jnp.arange(MP) < MN).astype(jnp.float32)
    rowsum = (sim * colmask[None, :]).sum(1)
    pos = jnp.exp((nh * nm).sum(-1) / T)
    denom = jnp.where(colmask > 0, rowsum - pos, 1.0)
    cl_rows = jnp.log(denom) - jnp.log(pos)
    cl = (cl_rows * colmask).sum() / MN
    return loss1 + cl * ALPHA


# speedup vs baseline: 10.2701x; 1.4375x over previous
import functools

import jax
import jax.numpy as jnp
from jax import lax
from jax.experimental import pallas as pl
from jax.experimental.pallas import tpu as pltpu
from jax.experimental.pallas import tpu_sc as plsc

N = 10000
E = 320000
D = 128
H = 256
T = 0.2
ALPHA = 0.5

NC, NS = 2, 16
NW = NC * NS
EPT = E // NW
KW = 125
NWIN = EPT // KW
HW = NWIN // 2
MN = 5000
MP = 5120
MPT = MP // NW
MKW = 80
NH = 10240
HPT = NH // NS
NA = 10240
APT = NA // NS
FL = 128

_MESH = dict(core_axis_name="c", subcore_axis_name="s")


def _wid():
    return lax.axis_index("s") * NC + lax.axis_index("c")


def _hist_body(src3, dst3, msk3, mupd3, ones_h, z_h,
               degs_o, degd_o, m01_o,
               sidx_v, didx_v, midx_v, mupd_v, ones_v, z_v, bounce_v,
               hs_sh, hd_sh, hm_sh):
    core = lax.axis_index("c")
    sid = lax.axis_index("s")
    wid = _wid()
    pltpu.sync_copy(z_h, z_v)
    pltpu.sync_copy(z_v, hs_sh.at[pl.ds(sid * HPT, HPT)])
    pltpu.sync_copy(z_v, hd_sh.at[pl.ds(sid * HPT, HPT)])
    pltpu.sync_copy(z_v, hm_sh.at[pl.ds(sid * HPT, HPT)])
    pltpu.sync_copy(ones_h, ones_v)
    pltpu.sync_copy(src3.at[wid], sidx_v)
    pltpu.sync_copy(dst3.at[wid], didx_v)
    pltpu.sync_copy(msk3.at[wid], midx_v)
    pltpu.sync_copy(mupd3.at[wid], mupd_v)
    plsc.subcore_barrier()

    def win(j, carry):
        pltpu.sync_copy(ones_v, hs_sh.at[sidx_v.at[j]], add=True)
        pltpu.sync_copy(ones_v, hd_sh.at[didx_v.at[j]], add=True)
        return carry

    lax.fori_loop(0, NWIN, win, 0)
    pltpu.sync_copy(mupd_v.at[0], hm_sh.at[midx_v.at[0]], add=True)
    pltpu.sync_copy(mupd_v.at[1], hm_sh.at[midx_v.at[1]], add=True)
    plsc.subcore_barrier()

    @pl.when(sid == 0)
    def _f0():
        pltpu.sync_copy(hs_sh, bounce_v)
        pltpu.sync_copy(bounce_v, degs_o.at[core])

    @pl.when(sid == 1)
    def _f1():
        pltpu.sync_copy(hd_sh, bounce_v)
        pltpu.sync_copy(bounce_v, degd_o.at[core])

    @pl.when(sid == 2)
    def _f2():
        pltpu.sync_copy(hm_sh, bounce_v)
        pltpu.sync_copy(bounce_v, m01_o.at[core])


@functools.cache
def _hist_kernel():
    return pl.kernel(
        _hist_body,
        out_type=(
            jax.ShapeDtypeStruct((NC, NH), jnp.float32),
            jax.ShapeDtypeStruct((NC, NH), jnp.float32),
            jax.ShapeDtypeStruct((NC, NH), jnp.float32),
        ),
        mesh=plsc.VectorSubcoreMesh(**_MESH),
        scratch_types=(
            pltpu.VMEM((NWIN, KW), jnp.int32),
            pltpu.VMEM((NWIN, KW), jnp.int32),
            pltpu.VMEM((MPT // MKW, MKW), jnp.int32),
            pltpu.VMEM((MPT // MKW, MKW), jnp.float32),
            pltpu.VMEM((KW,), jnp.float32),
            pltpu.VMEM((HPT,), jnp.float32),
            pltpu.VMEM((NH,), jnp.float32),
            pltpu.VMEM_SHARED((NH,), jnp.float32),
            pltpu.VMEM_SHARED((NH,), jnp.float32),
            pltpu.VMEM_SHARED((NH,), jnp.float32),
        ),
    )


def _make_prop(nchunks, ngather):
    def body(*refs):
        ys = refs[:nchunks]
        src3, dst3, z_h = refs[nchunks:nchunks + 3]
        k = nchunks + 3
        gidx_h = None
        gts = ()
        if ngather:
            gidx_h = refs[k]
            gts = refs[k + 1:k + 1 + ngather]
            k += 1 + ngather
        outs = refs[k:k + nchunks]
        k += nchunks
        gouts = refs[k:k + ngather]
        k += ngather
        sidx_v, didx_v, wbuf0_v, wbuf1_v, sem0, sem1 = refs[k:k + 6]
        if ngather:
            gidx_v = refs[k + 6]
        acc_sh = refs[-1]

        core = lax.axis_index("c")
        sid = lax.axis_index("s")
        wid = _wid()
        b0 = wbuf0_v.at[pl.ds(0, KW)]
        b1 = wbuf1_v.at[pl.ds(0, KW)]
        for c in range(nchunks):
            pltpu.sync_copy(z_h, wbuf0_v)
            for r in range(APT // FL):
                pltpu.sync_copy(
                    wbuf0_v, acc_sh.at[pl.ds(sid * APT + r * FL, FL)])
            plsc.subcore_barrier()
            for half in range(NWIN // HW):
                pltpu.sync_copy(src3.at[wid].at[pl.ds(half * HW, HW)], sidx_v)
                pltpu.sync_copy(dst3.at[wid].at[pl.ds(half * HW, HW)], didx_v)
                pltpu.async_copy(ys[c].at[sidx_v.at[0]], b0, sem0)

                def pair(i, carry):
                    j0 = 2 * i
                    pltpu.async_copy(ys[c].at[sidx_v.at[j0 + 1]], b1, sem1)
                    pltpu.make_async_copy(
                        ys[c].at[sidx_v.at[j0]], b0, sem0).wait()
                    pltpu.sync_copy(b0, acc_sh.at[didx_v.at[j0]], add=True)

                    @pl.when(i < HW // 2 - 1)
                    def _nx():
                        pltpu.async_copy(
                            ys[c].at[sidx_v.at[j0 + 2]], b0, sem0)

                    pltpu.make_async_copy(
                        ys[c].at[sidx_v.at[j0 + 1]], b1, sem1).wait()
                    pltpu.sync_copy(b1, acc_sh.at[didx_v.at[j0 + 1]], add=True)
                    return carry

                lax.fori_loop(0, HW // 2, pair, 0)
            plsc.subcore_barrier()
            for r in range(APT // FL):
                rows = pl.ds(sid * APT + r * FL, FL)
                pltpu.sync_copy(acc_sh.at[rows], wbuf0_v)
                pltpu.sync_copy(wbuf0_v, outs[c].at[core].at[rows])
            plsc.subcore_barrier()
        if ngather:
            pltpu.sync_copy(gidx_h.at[pl.ds(wid * MPT, MPT)], gidx_v)
            g0 = wbuf0_v.at[pl.ds(0, MKW)]
            g1 = wbuf1_v.at[pl.ds(0, MKW)]
            gsrcs = [gts[t].at[gidx_v.at[pl.ds(j * MKW, MKW)]]
                     for t in range(ngather) for j in range(MPT // MKW)]
            gdsts = [gouts[t].at[pl.ds(wid * MPT + j * MKW, MKW)]
                     for t in range(ngather) for j in range(MPT // MKW)]
            bufs = [g0, g1]
            sems = [sem0, sem1]
            pltpu.async_copy(gsrcs[0], bufs[0], sems[0])
            for i in range(len(gsrcs)):
                if i + 1 < len(gsrcs):
                    pltpu.async_copy(
                        gsrcs[i + 1], bufs[(i + 1) % 2], sems[(i + 1) % 2])
                pltpu.make_async_copy(gsrcs[i], bufs[i % 2], sems[i % 2]).wait()
                pltpu.sync_copy(bufs[i % 2], gdsts[i])

    out_type = tuple(
        jax.ShapeDtypeStruct((NC, NA, D), jnp.float32) for _ in range(nchunks)
    ) + tuple(
        jax.ShapeDtypeStruct((MP, D), jnp.float32) for _ in range(ngather)
    )
    scratch = [
        pltpu.VMEM((HW, KW), jnp.int32),
        pltpu.VMEM((HW, KW), jnp.int32),
        pltpu.VMEM((FL, D), jnp.float32),
        pltpu.VMEM((FL, D), jnp.float32),
        pltpu.SemaphoreType.DMA,
        pltpu.SemaphoreType.DMA,
    ]
    if ngather:
        scratch.append(pltpu.VMEM((MPT,), jnp.int32))
    scratch.append(pltpu.VMEM_SHARED((NA, D), jnp.float32))
    return pl.kernel(
        body,
        out_type=out_type,
        mesh=plsc.VectorSubcoreMesh(**_MESH),
        scratch_types=tuple(scratch),
    )


_make_prop = functools.cache(_make_prop)


def _hist_call(*args):
    return _hist_kernel()(*args)


def _prop2(*args):
    return _make_prop(2, 0)(*args)


def _prop1g2(*args):
    return _make_prop(1, 2)(*args)


def _bn(x, g, b):
    m = x.mean(0)
    v = x.var(0)
    return (x - m) / jnp.sqrt(v + 1e-5) * g + b


def _prelu(x, a):
    return jnp.where(x >= 0, x, a * x)


def _l2n(x):
    return x / jnp.maximum(jnp.linalg.norm(x, axis=-1, keepdims=True), 1e-12)


def kernel(feat, edge_index, mask_nodes, W1, b1, g1, be1, a1, W2, b2, g2, be2,
           a2, tW1, tb1, tg1, tbe1, ta1, tW2, tb2, tg2, tbe2, ta2,
           dW, db, dg, dbe, da, mask_token,
           pW1, pb1, pW2, pb2, qW1, qb1, qW2, qb2):
    src3 = edge_index[0].reshape(NW, NWIN, KW)
    dst3 = edge_index[1].reshape(NW, NWIN, KW)
    mpad = jnp.concatenate(
        [mask_nodes, jnp.zeros((MP - MN,), jnp.int32)])
    msk3 = mpad.reshape(NW, MPT // MKW, MKW)
    mupd3 = jnp.concatenate(
        [jnp.ones((MN,), jnp.float32), jnp.zeros((MP - MN,), jnp.float32)]
    ).reshape(NW, MPT // MKW, MKW)
    ones_h = jnp.ones((KW,), jnp.float32)
    zh_h = jnp.zeros((HPT,), jnp.float32)
    zr_h = jnp.zeros((FL, D), jnp.float32)

    degs2, degd2, m012 = _hist_call(src3, dst3, msk3, mupd3, ones_h, zh_h)
    ns = jnp.clip(degs2[0, :N] + degs2[1, :N], 1.0, None) ** -0.5
    nd = jnp.clip(degd2[0, :N] + degd2[1, :N], 1.0, None) ** -0.5
    m01 = (m012[0, :N] + m012[1, :N])[:, None]

    x = feat * (1.0 - m01) + m01 * mask_token
    y0 = x * ns[:, None]
    y1 = feat * ns[:, None]
    p0, p1 = _prop2(y0, y1, src3, dst3, zr_h)

    def post(p, W, b, g, be, a):
        z = nd[:, None] * (p[0, :N] + p[1, :N])
        return _prelu(_bn(z @ W + b, g, be), a)

    e1 = post(p0, W1, b1, g1, be1, a1)
    te1 = post(p1, tW1, tb1, tg1, tbe1, ta1)

    y2a = (e1 * ns[:, None]) @ W2
    y2b = (te1 * ns[:, None]) @ tW2
    q0, q1 = _prop2(y2a, y2b, src3, dst3, zr_h)

    def post2(p, b, g, be, a):
        z = nd[:, None] * (p[0, :N] + p[1, :N])
        return _prelu(_bn(z + b, g, be), a)

    o = post2(q0, b2, g2, be2, a2)
    h2 = post2(q1, tb2, tg2, tbe2, ta2)

    y3 = ((o * (1.0 - m01)) * ns[:, None]) @ dW
    r0, om, hm = _prop1g2(y3, src3, dst3, zr_h, mpad, o, h2)
    re_x = _prelu(_bn(nd[:, None] * (r0[0, :N] + r0[1, :N]) + db, dg, dbe), da)

    rn = _l2n(re_x)
    fn = _l2n(feat)
    cos = (rn * fn).sum(-1)
    loss1 = (m01[:, 0] * (1.0 - cos)).sum() / MN

    ch = jax.nn.relu(hm @ pW1 + pb1) @ pW2 + pb2
    cm = jax.nn.relu(om @ qW1 + qb1) @ qW2 + qb2
    nh = _l2n(ch)
    nm = _l2n(cm)
    sim = jnp.exp((nh @ nm.T) / T)
    colmask = (jnp.arange(MP) < MN).astype(jnp.float32)
    rowsum = (sim * colmask[None, :]).sum(1)
    pos = jnp.exp((nh * nm).sum(-1) / T)
    denom = jnp.where(colmask > 0, rowsum - pos, 1.0)
    cl_rows = jnp.log(denom) - jnp.log(pos)
    cl = (cl_rows * colmask).sum() / MN
    return loss1 + cl * ALPHA
